# Initial kernel scaffold; baseline (speedup 1.0000x reference)
import jax
import jax.numpy as jnp
from jax.experimental import pallas as pl


def kernel(field_indices, continuous_features, embedding, linear_emb, W1, b1, g1, be1, W2, b2, g2, be2, W3, b3, g3, be3, W4, b4):
    raise NotImplementedError("write your pallas kernel here")


# speedup vs baseline: 7.4159x; 7.4159x over previous
import functools

import jax
import jax.numpy as jnp
from jax import lax
from jax.experimental import pallas as pl
from jax.experimental.pallas import tpu as pltpu
from jax.experimental.pallas import tpu_sc as plsc

B = 16384
F = 26
V = 1000
D = 128
NCF = 4
ROWS = B * F

SC_CORES = 2
SC_SUBCORES = 16
NW = SC_CORES * SC_SUBCORES
ROWS_PER_W = ROWS // NW
CH = 512
CHB = CH // 128
NCHUNK = ROWS_PER_W // CH


def _sc_gather(fi2, emb):
    mesh = plsc.VectorSubcoreMesh(core_axis_name="c", subcore_axis_name="s")

    @functools.partial(
        pl.kernel,
        mesh=mesh,
        out_type=jax.ShapeDtypeStruct((ROWS, D), jnp.float32),
        scratch_types=[
            pltpu.VMEM((CHB, 128), jnp.int32),
            pltpu.VMEM((CH, D), jnp.float32),
            pltpu.SemaphoreType.DMA,
        ],
    )
    def k(fi_hbm, emb_hbm, oute_hbm, idx_v, rows_v, sem_e):
        wid = lax.axis_index("c") * SC_SUBCORES + lax.axis_index("s")
        w_base = wid * ROWS_PER_W
        w_row0 = wid * (ROWS_PER_W // 128)

        def chunk_body(ci, carry):
            base = w_base + ci * CH
            rb = w_row0 + ci * CHB
            pltpu.sync_copy(fi_hbm.at[pl.ds(rb, CHB)], idx_v)
            for j in range(CHB):
                for k16 in range(8):
                    p = base + j * 128 + k16 * 16 + lax.iota(jnp.int32, 16)
                    off = lax.rem(p, F) * V
                    sl = (j, pl.ds(k16 * 16, 16))
                    idx_v[sl] = idx_v[sl] + off
            copies = [
                pltpu.make_async_copy(
                    emb_hbm.at[idx_v.at[j]],
                    rows_v.at[pl.ds(j * 128, 128)], sem_e)
                for j in range(CHB)
            ]
            for c in copies:
                c.start()
            for c in copies:
                c.wait()
            pltpu.sync_copy(rows_v, oute_hbm.at[pl.ds(base, CH)])
            return carry

        lax.fori_loop(0, NCHUNK, chunk_body, 0)

    return k(fi2, emb)


LCH = 512
LCHB = LCH // 128
LNCHUNK = ROWS_PER_W // LCH


def _sc_linear(fi_flat, lin_flat):
    mesh = plsc.VectorSubcoreMesh(core_axis_name="c", subcore_axis_name="s")

    @functools.partial(
        pl.kernel,
        mesh=mesh,
        out_type=jax.ShapeDtypeStruct((ROWS,), jnp.float32),
        scratch_types=[
            pltpu.VMEM((F * V,), jnp.float32),
            pltpu.VMEM((LCH,), jnp.int32),
            pltpu.VMEM((LCH,), jnp.float32),
        ],
        compiler_params=pltpu.CompilerParams(needs_layout_passes=False),
    )
    def k(fi_hbm, lin_hbm, outl_hbm, tab_v, idx_v, val_v):
        wid = lax.axis_index("c") * SC_SUBCORES + lax.axis_index("s")
        w_base = wid * ROWS_PER_W
        pltpu.sync_copy(lin_hbm, tab_v)

        def chunk_body(ci, carry):
            base = w_base + ci * LCH
            pltpu.sync_copy(fi_hbm.at[pl.ds(base, LCH)], idx_v)
            for j in range(LCH // 16):
                p = base + j * 16 + lax.iota(jnp.int32, 16)
                sl = pl.ds(j * 16, 16)
                gi = idx_v[sl] + lax.rem(p, F) * V
                val_v[sl] = plsc.load_gather(tab_v, [gi])
            pltpu.sync_copy(val_v, outl_hbm.at[pl.ds(base, LCH)])
            return carry

        lax.fori_loop(0, LNCHUNK, chunk_body, 0)

    return k(fi_flat, lin_flat)


def _ln(h, g, b):
    m = jnp.mean(h, axis=-1, keepdims=True)
    d = h - m
    v = jnp.mean(d * d, axis=-1, keepdims=True)
    return d * lax.rsqrt(v + 1e-5) * g[None, :] + b[None, :]


def _tc_body(xe_ref, cont_ref, lv_ref, w1e_ref, w1c_ref, b1_ref, g1_ref,
             be1_ref, w2_ref, b2_ref, g2_ref, be2_ref, w3_ref, b3_ref, g3_ref,
             be3_ref, w4_ref, b4_ref, out_ref):
    x = xe_ref[...]
    s = x[:, 0:D]
    ss = s * s
    for f in range(1, F):
        sl = x[:, f * D:(f + 1) * D]
        s = s + sl
        ss = ss + sl * sl
    second = 0.5 * jnp.sum(s * s - ss, axis=1, keepdims=True)
    first = jnp.sum(lv_ref[...], axis=1, keepdims=True)
    h = jnp.dot(x, w1e_ref[...], preferred_element_type=jnp.float32)
    h = h + jnp.dot(cont_ref[...], w1c_ref[...],
                    preferred_element_type=jnp.float32)
    h = jnp.maximum(h + b1_ref[...][None, :], 0.0)
    h = _ln(h, g1_ref[...], be1_ref[...])
    h = jnp.dot(h, w2_ref[...], preferred_element_type=jnp.float32)
    h = jnp.maximum(h + b2_ref[...][None, :], 0.0)
    h = _ln(h, g2_ref[...], be2_ref[...])
    h = jnp.dot(h, w3_ref[...], preferred_element_type=jnp.float32)
    h = jnp.maximum(h + b3_ref[...][None, :], 0.0)
    h = _ln(h, g3_ref[...], be3_ref[...])
    deep = jnp.dot(h, w4_ref[...], preferred_element_type=jnp.float32)
    out_ref[...] = first + second + deep + b4_ref[0]


def _tc_mlp(xe, cont, lv, w1e, w1c, b1, g1, be1, w2, b2, g2, be2, w3, b3, g3,
            be3, w4, b4):
    BB = 256
    grid = (B // BB,)
    row = lambda i: (i, 0)
    rep2 = lambda i: (0, 0)
    rep1 = lambda i: (0,)
    h1, h2, h3 = 1024, 512, 256
    return pl.pallas_call(
        _tc_body,
        grid=grid,
        in_specs=[
            pl.BlockSpec((BB, F * D), row),
            pl.BlockSpec((BB, NCF), row),
            pl.BlockSpec((BB, F), row),
            pl.BlockSpec((F * D, h1), rep2),
            pl.BlockSpec((NCF, h1), rep2),
            pl.BlockSpec((h1,), rep1),
            pl.BlockSpec((h1,), rep1),
            pl.BlockSpec((h1,), rep1),
            pl.BlockSpec((h1, h2), rep2),
            pl.BlockSpec((h2,), rep1),
            pl.BlockSpec((h2,), rep1),
            pl.BlockSpec((h2,), rep1),
            pl.BlockSpec((h2, h3), rep2),
            pl.BlockSpec((h3,), rep1),
            pl.BlockSpec((h3,), rep1),
            pl.BlockSpec((h3,), rep1),
            pl.BlockSpec((h3, 1), rep2),
            pl.BlockSpec((1,), rep1),
        ],
        out_specs=pl.BlockSpec((BB, 1), row),
        out_shape=jax.ShapeDtypeStruct((B, 1), jnp.float32),
        compiler_params=pltpu.CompilerParams(
            dimension_semantics=("arbitrary",)),
    )(xe, cont, lv, w1e, w1c, b1, g1, be1, w2, b2, g2, be2, w3, b3, g3, be3,
      w4, b4)


def kernel(field_indices, continuous_features, embedding, linear_emb, W1, b1,
           g1, be1, W2, b2, g2, be2, W3, b3, g3, be3, W4, b4):
    fi2 = field_indices.astype(jnp.int32).reshape(ROWS // 128, 128)
    rows = _sc_gather(fi2, embedding)
    linvals = _sc_linear(fi2.reshape(-1), linear_emb.reshape(-1))
    embeds = rows.reshape(B, F, D)
    xe = rows.reshape(B, F * D)
    lv = linvals.reshape(B, F)
    w1e = W1[:F * D]
    w1c = W1[F * D:]
    logits = _tc_mlp(xe, continuous_features, lv, w1e, w1c, b1, g1, be1, W2,
                     b2, g2, be2, W3, b3, g3, be3, W4, b4)
    return (logits, embeds)


# speedup vs baseline: 8.4616x; 1.1410x over previous
import functools

import jax
import jax.numpy as jnp
from jax import lax
from jax.experimental import pallas as pl
from jax.experimental.pallas import tpu as pltpu
from jax.experimental.pallas import tpu_sc as plsc

B = 16384
LOG2B = 14
F = 26
V = 1000
D = 128
NCF = 4
ROWS = B * F

SC_CORES = 2
SC_SUBCORES = 16
NW = SC_CORES * SC_SUBCORES
ROWS_PER_W = ROWS // NW
CH = 512
CHB = CH // 128
NCHUNK = ROWS_PER_W // CH


def _sc_gather(fit2, emb):
    mesh = plsc.VectorSubcoreMesh(core_axis_name="c", subcore_axis_name="s")

    @functools.partial(
        pl.kernel,
        mesh=mesh,
        out_type=jax.ShapeDtypeStruct((ROWS, D), jnp.float32),
        scratch_types=[
            pltpu.VMEM((CHB, 128), jnp.int32),
            pltpu.VMEM((CH, D), jnp.float32),
            pltpu.SemaphoreType.DMA,
        ],
    )
    def k(fi_hbm, emb_hbm, oute_hbm, idx_v, rows_v, sem_e):
        wid = lax.axis_index("c") * SC_SUBCORES + lax.axis_index("s")
        w_base = wid * ROWS_PER_W
        w_row0 = wid * (ROWS_PER_W // 128)

        def chunk_body(ci, carry):
            base = w_base + ci * CH
            rb = w_row0 + ci * CHB
            pltpu.sync_copy(fi_hbm.at[pl.ds(rb, CHB)], idx_v)
            for j in range(CHB):
                for k16 in range(8):
                    p = base + j * 128 + k16 * 16 + lax.iota(jnp.int32, 16)
                    off = lax.shift_right_logical(p, LOG2B) * V
                    sl = (j, pl.ds(k16 * 16, 16))
                    idx_v[sl] = idx_v[sl] + off
            copies = [
                pltpu.make_async_copy(
                    emb_hbm.at[idx_v.at[j]],
                    rows_v.at[pl.ds(j * 128, 128)], sem_e)
                for j in range(CHB)
            ]
            for c in copies:
                c.start()
            for c in copies:
                c.wait()
            pltpu.sync_copy(rows_v, oute_hbm.at[pl.ds(base, CH)])
            return carry

        lax.fori_loop(0, NCHUNK, chunk_body, 0)

    return k(fit2, emb)


LCH = 512
LNCHUNK = ROWS_PER_W // LCH


def _sc_linear(fi_flat, lin_flat):
    mesh = plsc.VectorSubcoreMesh(core_axis_name="c", subcore_axis_name="s")

    @functools.partial(
        pl.kernel,
        mesh=mesh,
        out_type=jax.ShapeDtypeStruct((ROWS,), jnp.float32),
        scratch_types=[
            pltpu.VMEM((F * V,), jnp.float32),
            pltpu.VMEM((LCH,), jnp.int32),
            pltpu.VMEM((LCH,), jnp.float32),
        ],
        compiler_params=pltpu.CompilerParams(needs_layout_passes=False),
    )
    def k(fi_hbm, lin_hbm, outl_hbm, tab_v, idx_v, val_v):
        wid = lax.axis_index("c") * SC_SUBCORES + lax.axis_index("s")
        w_base = wid * ROWS_PER_W
        pltpu.sync_copy(lin_hbm, tab_v)

        def chunk_body(ci, carry):
            base = w_base + ci * LCH
            pltpu.sync_copy(fi_hbm.at[pl.ds(base, LCH)], idx_v)
            for j in range(LCH // 16):
                p = base + j * 16 + lax.iota(jnp.int32, 16)
                sl = pl.ds(j * 16, 16)
                gi = idx_v[sl] + lax.shift_right_logical(p, LOG2B) * V
                val_v[sl] = plsc.load_gather(tab_v, [gi])
            pltpu.sync_copy(val_v, outl_hbm.at[pl.ds(base, LCH)])
            return carry

        lax.fori_loop(0, LNCHUNK, chunk_body, 0)

    return k(fi_flat, lin_flat)


def _ln(h, g, b):
    m = jnp.mean(h, axis=-1, keepdims=True)
    d = h - m
    v = jnp.mean(d * d, axis=-1, keepdims=True)
    return d * lax.rsqrt(v + 1e-5) * g[None, :] + b[None, :]


def _tc_body(x3_ref, cont_ref, lv_ref, w1e_ref, w1c_ref, b1_ref, g1_ref,
             be1_ref, w2_ref, b2_ref, g2_ref, be2_ref, w3_ref, b3_ref, g3_ref,
             be3_ref, w4_ref, b4_ref, out_ref, emb_ref):
    x3 = x3_ref[...]
    xf = x3[0]
    emb_ref[:, 0, :] = xf
    s = xf
    ss = xf * xf
    h = jnp.dot(xf, w1e_ref[0:D, :], preferred_element_type=jnp.float32)
    for f in range(1, F):
        xf = x3[f]
        emb_ref[:, f, :] = xf
        s = s + xf
        ss = ss + xf * xf
        h = h + jnp.dot(xf, w1e_ref[f * D:(f + 1) * D, :],
                        preferred_element_type=jnp.float32)
    second = 0.5 * jnp.sum(s * s - ss, axis=1, keepdims=True)
    first = jnp.sum(lv_ref[...], axis=0)[:, None]
    h = h + jnp.dot(cont_ref[...], w1c_ref[...],
                    preferred_element_type=jnp.float32)
    h = jnp.maximum(h + b1_ref[...][None, :], 0.0)
    h = _ln(h, g1_ref[...], be1_ref[...])
    h = jnp.dot(h, w2_ref[...], preferred_element_type=jnp.float32)
    h = jnp.maximum(h + b2_ref[...][None, :], 0.0)
    h = _ln(h, g2_ref[...], be2_ref[...])
    h = jnp.dot(h, w3_ref[...], preferred_element_type=jnp.float32)
    h = jnp.maximum(h + b3_ref[...][None, :], 0.0)
    h = _ln(h, g3_ref[...], be3_ref[...])
    deep = jnp.dot(h, w4_ref[...], preferred_element_type=jnp.float32)
    out_ref[...] = first + second + deep + b4_ref[0]


def _tc_mlp(x3, cont, lv, w1e, w1c, b1, g1, be1, w2, b2, g2, be2, w3, b3, g3,
            be3, w4, b4):
    BB = 256
    grid = (B // BB,)
    row = lambda i: (i, 0)
    mid = lambda i: (0, i, 0)
    rep2 = lambda i: (0, 0)
    rep1 = lambda i: (0,)
    h1, h2, h3 = 1024, 512, 256
    return pl.pallas_call(
        _tc_body,
        grid=grid,
        in_specs=[
            pl.BlockSpec((F, BB, D), mid),
            pl.BlockSpec((BB, NCF), row),
            pl.BlockSpec((F, BB), lambda i: (0, i)),
            pl.BlockSpec((F * D, h1), rep2),
            pl.BlockSpec((NCF, h1), rep2),
            pl.BlockSpec((h1,), rep1),
            pl.BlockSpec((h1,), rep1),
            pl.BlockSpec((h1,), rep1),
            pl.BlockSpec((h1, h2), rep2),
            pl.BlockSpec((h2,), rep1),
            pl.BlockSpec((h2,), rep1),
            pl.BlockSpec((h2,), rep1),
            pl.BlockSpec((h2, h3), rep2),
            pl.BlockSpec((h3,), rep1),
            pl.BlockSpec((h3,), rep1),
            pl.BlockSpec((h3,), rep1),
            pl.BlockSpec((h3, 1), rep2),
            pl.BlockSpec((1,), rep1),
        ],
        out_specs=[
            pl.BlockSpec((BB, 1), row),
            pl.BlockSpec((BB, F, D), lambda i: (i, 0, 0)),
        ],
        out_shape=[
            jax.ShapeDtypeStruct((B, 1), jnp.float32),
            jax.ShapeDtypeStruct((B, F, D), jnp.float32),
        ],
        compiler_params=pltpu.CompilerParams(
            dimension_semantics=("arbitrary",)),
    )(x3, cont, lv, w1e, w1c, b1, g1, be1, w2, b2, g2, be2, w3, b3, g3, be3,
      w4, b4)


def kernel(field_indices, continuous_features, embedding, linear_emb, W1, b1,
           g1, be1, W2, b2, g2, be2, W3, b3, g3, be3, W4, b4):
    fit = field_indices.astype(jnp.int32).T
    fit2 = fit.reshape(ROWS // 128, 128)
    rows = _sc_gather(fit2, embedding)
    linvals = _sc_linear(fit.reshape(-1), linear_emb.reshape(-1))
    x3 = rows.reshape(F, B, D)
    lv = linvals.reshape(F, B)
    w1e = W1[:F * D]
    w1c = W1[F * D:]
    logits, embeds = _tc_mlp(x3, continuous_features, lv, w1e, w1c, b1, g1,
                             be1, W2, b2, g2, be2, W3, b3, g3, be3, W4, b4)
    return (logits, embeds)
